# Initial kernel scaffold; baseline (speedup 1.0000x reference)
import functools

import jax
import jax.numpy as jnp
from jax import lax
from jax.experimental import pallas as pl
from jax.experimental.pallas import tpu as pltpu
from jax.experimental.pallas import tpu_sc as plsc

_HEADS = 8
_HID = 128
_MOL = 512
_PRO = 4096
_E = 131072
_B = 32

_NC = 2
_NS = 16
_CHUNK = _E // _NS
_NGRP = _CHUNK // 16
_NIDX = _CHUNK // 128
_NSLOT = _MOL * _MOL
_ROWS_PER_TILE = _NSLOT // _NS


def _prep_body(mol_ref, pro_ref, w1a_ref, w1b_ref, a_ref, p_ref):
    a_ref[...] = jnp.dot(mol_ref[...], w1a_ref[...],
                         preferred_element_type=jnp.float32)
    p_ref[...] = jnp.dot(pro_ref[...], w1b_ref[...],
                         preferred_element_type=jnp.float32)


def _prep(mol, pro512, w1a, w1b):
    return pl.pallas_call(
        _prep_body,
        out_shape=[
            jax.ShapeDtypeStruct((_MOL, _HEADS), jnp.float32),
            jax.ShapeDtypeStruct((_MOL, _HEADS), jnp.float32),
        ],
    )(mol, pro512, w1a, w1b)


_sc_mesh = plsc.VectorSubcoreMesh(
    core_axis_name="c", subcore_axis_name="s",
    num_cores=_NC, num_subcores=_NS)


@functools.partial(
    pl.kernel,
    out_type=jax.ShapeDtypeStruct((_HEADS, _MOL, _MOL), jnp.float32),
    mesh=_sc_mesh,
    scratch_types=[
        pltpu.VMEM((_CHUNK,), jnp.int32),
        pltpu.VMEM((_CHUNK,), jnp.int32),
        pltpu.VMEM((_CHUNK,), jnp.float32),
        pltpu.VMEM((_NIDX, 128), jnp.int32),
        pltpu.VMEM((_CHUNK, 4), jnp.float32),
        pltpu.VMEM((_MOL * _HEADS,), jnp.float32),
        pltpu.VMEM((_MOL * _HEADS,), jnp.float32),
        pltpu.VMEM((_HEADS, 16), jnp.float32),
        pltpu.VMEM((16, _MOL), jnp.float32),
        pltpu.VMEM_SHARED((_NSLOT, 4), jnp.float32),
    ],
)
def _sc_scatter(src_hbm, dst_hbm, dist_hbm, a_hbm, p_hbm, b1_hbm, z_hbm,
                out_hbm,
                src_v, dst_v, dist_v, idx_v, hbuf, a_v, p_v, b1_v, plane_v,
                acc):
    c = lax.axis_index("c")
    s = lax.axis_index("s")
    base = s * _CHUNK
    hbase = c * 4

    pltpu.sync_copy(src_hbm.at[pl.ds(base, _CHUNK)], src_v)
    pltpu.sync_copy(dst_hbm.at[pl.ds(base, _CHUNK)], dst_v)
    pltpu.sync_copy(dist_hbm.at[pl.ds(base, _CHUNK)], dist_v)
    pltpu.sync_copy(a_hbm, a_v)
    pltpu.sync_copy(p_hbm, p_v)
    pltpu.sync_copy(b1_hbm, b1_v)
    pltpu.sync_copy(z_hbm, acc.at[pl.ds(s * _ROWS_PER_TILE, _ROWS_PER_TILE)])
    plsc.subcore_barrier()

    iota16 = lax.iota(jnp.int32, 16)
    b1r = [b1_v[hbase + hl] for hl in range(4)]

    def group(k, carry):
        sl = pl.ds(k * 16, 16)
        s16 = src_v[sl]
        d16 = dst_v[sl]
        w16 = dist_v[sl]
        idx_v[k // 8, pl.ds((k % 8) * 16, 16)] = s16 * _MOL + d16
        e16 = k * 16 + iota16
        for hl in range(4):
            h = hbase + hl
            av = plsc.load_gather(a_v, [s16 * _HEADS + h])
            pv = plsc.load_gather(p_v, [d16 * _HEADS + h])
            x = w16 * (av + pv) + b1r[hl]
            hv = jnp.where(x > 0.0, x + 1.0, jnp.exp(x))
            plsc.store_scatter(hbuf, [e16, jnp.full((16,), hl, jnp.int32)], hv)
        return carry

    lax.fori_loop(0, _NGRP, group, 0)

    def scat(j, carry):
        pltpu.sync_copy(hbuf.at[pl.ds(j * 128, 128)],
                        acc.at[idx_v.at[j]], add=True)
        return carry

    lax.fori_loop(0, _NIDX, scat, 0)
    plsc.subcore_barrier()

    for half in range(2):
        row0 = s * _ROWS_PER_TILE + half * (_ROWS_PER_TILE // 2)
        pltpu.sync_copy(acc.at[pl.ds(row0, _ROWS_PER_TILE // 2)], hbuf)
        for hl in range(4):
            hsel = jnp.full((16,), hl, jnp.int32)

            def deint(g, carry):
                slot16 = g * 16 + iota16
                vals = plsc.load_gather(hbuf, [slot16, hsel])
                plane_v[g // 32, pl.ds((g % 32) * 16, 16)] = vals
                return carry

            lax.fori_loop(0, (_ROWS_PER_TILE // 2) // 16, deint, 0)
            srow = s * 32 + half * 16
            pltpu.sync_copy(
                plane_v,
                out_hbm.at[hbase + hl, pl.ds(srow, 16), :])


def _asm_body(acc_ref, pb_ref, w3_ref, b3_ref, out_ref, pred_ref, yscr):
    h = pl.program_id(0)
    x = acc_ref[0]
    out_ref[0, :, 0:_MOL] = x
    out_ref[0, :, _MOL:] = jnp.zeros((_MOL, _PRO - _MOL), jnp.float32)
    yscr[h, :] = jnp.sum(x, axis=0)

    @pl.when(h == _HEADS - 1)
    def _():
        pb = pb_ref[0]
        oh = (pb[:, None] == lax.broadcasted_iota(
            jnp.int32, (_MOL, _B), 1)).astype(jnp.float32)
        yb = jnp.dot(yscr[...], oh,
                     preferred_element_type=jnp.float32)
        pred = 0.01 * jnp.dot(yb.T, w3_ref[...],
                              preferred_element_type=jnp.float32)
        pred_ref[...] = pred + b3_ref[...]


def _assemble(acc8, pb512, w3, b3):
    return pl.pallas_call(
        _asm_body,
        grid=(_HEADS,),
        in_specs=[
            pl.BlockSpec((1, _MOL, _MOL), lambda h: (h, 0, 0)),
            pl.BlockSpec((1, _MOL), lambda h: (0, 0)),
            pl.BlockSpec((_HEADS, 1), lambda h: (0, 0)),
            pl.BlockSpec((1, 1), lambda h: (0, 0)),
        ],
        out_specs=[
            pl.BlockSpec((1, _MOL, _PRO), lambda h: (h, 0, 0)),
            pl.BlockSpec((_B, 1), lambda h: (0, 0)),
        ],
        out_shape=[
            jax.ShapeDtypeStruct((_HEADS, _MOL, _PRO), jnp.float32),
            jax.ShapeDtypeStruct((_B, 1), jnp.float32),
        ],
        scratch_shapes=[pltpu.VMEM((_HEADS, _MOL), jnp.float32)],
    )(acc8, pb512, w3, b3)


def kernel(mol_feats, pro_feats, pro_batch, bipartite_edge_index,
           bipartite_edge_attr, W1, b1, W3, b3):
    src = bipartite_edge_index[0]
    dst = bipartite_edge_index[1]
    dist = bipartite_edge_attr[:, 0]

    a_tab, p_tab = _prep(mol_feats, pro_feats[:_MOL],
                         W1[:_HID], W1[_HID:])
    b1_bc = jnp.broadcast_to(b1[:, None], (_HEADS, 16))
    zrows = jnp.zeros((_ROWS_PER_TILE, 4), jnp.float32)

    acc8 = _sc_scatter(src, dst, dist,
                       a_tab.reshape(-1), p_tab.reshape(-1), b1_bc, zrows)

    inter, pred = _assemble(acc8, pro_batch[:_MOL].reshape(1, _MOL),
                            W3, b3.reshape(1, 1))
    return (inter, pred)


# speedup vs baseline: 10.0798x; 10.0798x over previous
import functools

import jax
import jax.numpy as jnp
from jax import lax
from jax.experimental import pallas as pl
from jax.experimental.pallas import tpu as pltpu
from jax.experimental.pallas import tpu_sc as plsc

_HEADS = 8
_HID = 128
_MOL = 512
_PRO = 4096
_E = 131072
_B = 32

_NC = 2
_NS = 16
_CHUNK = _E // _NS
_PASSES = 8
_PE = _CHUNK // _PASSES
_NGRP = _PE // 16
_NIDX = _PE // 128
_AROWS = _MOL * _MOL // 4
_ARPT = _AROWS // _NS
_NPART = _ARPT // _PE


def _prep_body(mol_ref, pro_ref, w1a_ref, w1b_ref, a_ref, p_ref):
    a_ref[...] = jnp.dot(mol_ref[...], w1a_ref[...],
                         preferred_element_type=jnp.float32)
    p_ref[...] = jnp.dot(pro_ref[...], w1b_ref[...],
                         preferred_element_type=jnp.float32)


def _prep(mol, pro512, w1a, w1b):
    return pl.pallas_call(
        _prep_body,
        out_shape=[
            jax.ShapeDtypeStruct((_MOL, _HEADS), jnp.float32),
            jax.ShapeDtypeStruct((_MOL, _HEADS), jnp.float32),
        ],
    )(mol, pro512, w1a, w1b)


def _sc_body(src_hbm, dst_hbm, dist_hbm, a_hbm, p_hbm, b1_hbm, z_hbm,
             out_hbm,
             src_v, dst_v, dist_v, idx_v, hbuf, a_v, p_v, b1_v, plane_v,
             acc):
    c = lax.axis_index("c")
    s = lax.axis_index("s")

    pltpu.sync_copy(z_hbm, hbuf)
    pltpu.sync_copy(a_hbm.at[c], a_v)
    pltpu.sync_copy(p_hbm.at[c], p_v)
    pltpu.sync_copy(b1_hbm.at[c], b1_v)

    iota16 = lax.iota(jnp.int32, 16)
    b1r = [b1_v[hl] for hl in range(4)]

    def fill_idx_seq(tile_base):
        def fillrow(j, carry):
            rowbase = tile_base + j * 128
            for col in range(8):
                idx_v[j, pl.ds(col * 16, 16)] = rowbase + col * 16 + iota16
            return carry
        lax.fori_loop(0, _NIDX, fillrow, 0)

    for q in range(_ARPT // _PE):
        fill_idx_seq(s * _ARPT + q * _PE)
        for j in range(_NIDX):
            pltpu.sync_copy(hbuf.at[pl.ds(0, 128)], acc.at[idx_v.at[j]])
    plsc.subcore_barrier()

    for p in range(_PASSES):
        base = s * _CHUNK + p * _PE
        pltpu.sync_copy(src_hbm.at[pl.ds(base, _PE)], src_v)
        pltpu.sync_copy(dst_hbm.at[pl.ds(base, _PE)], dst_v)
        pltpu.sync_copy(dist_hbm.at[pl.ds(base, _PE)], dist_v)
        if p > 0:
            pltpu.sync_copy(z_hbm, hbuf)

        def group(k, carry):
            sl = pl.ds(k * 16, 16)
            s16 = src_v[sl]
            d16 = dst_v[sl]
            w16 = dist_v[sl]
            idx_v[k // 8, pl.ds((k % 8) * 16, 16)] = s16 * 128 + (d16 >> 2)
            e16 = k * 16 + iota16
            cbase = (d16 & 3) * 4
            for hl in range(4):
                av = plsc.load_gather(a_v, [s16 * 4 + hl])
                pv = plsc.load_gather(p_v, [d16 * 4 + hl])
                x = w16 * (av + pv) + b1r[hl]
                hv = jnp.where(x > 0.0, x + 1.0, jnp.exp(x))
                plsc.store_scatter(hbuf, [e16, cbase + hl], hv)
            return carry

        lax.fori_loop(0, _NGRP, group, 0)

        for j in range(_NIDX):
            pltpu.sync_copy(hbuf.at[pl.ds(j * 128, 128)],
                            acc.at[idx_v.at[j]], add=True)

    plsc.subcore_barrier()

    iotad4 = iota16 // 4
    for part in range(_NPART):
        fill_idx_seq(s * _ARPT + part * _PE)
        for j in range(_NIDX):
            pltpu.sync_copy(acc.at[idx_v.at[j]],
                            hbuf.at[pl.ds(j * 128, 128)])
        for hl in range(4):
            colh = (iota16 & 3) * 4 + hl

            def deint(g, carry):
                row16 = (g // 32) * 128 + (g % 32) * 4 + iotad4
                vals = plsc.load_gather(hbuf, [row16, colh])
                plane_v[g // 32, pl.ds((g % 32) * 16, 16)] = vals
                return carry

            lax.fori_loop(0, (_PE // 128) * _MOL // 16, deint, 0)
            srow = s * 32 + part * (_PE * 4 // _MOL)
            pltpu.sync_copy(
                plane_v,
                out_hbm.at[c * 4 + hl, pl.ds(srow, _PE * 4 // _MOL), :])


@functools.cache
def _sc_scatter_fn():
    mesh = plsc.VectorSubcoreMesh(
        core_axis_name="c", subcore_axis_name="s",
        num_cores=_NC, num_subcores=_NS)
    return pl.kernel(
        _sc_body,
        out_type=jax.ShapeDtypeStruct((_HEADS, _MOL, _MOL), jnp.float32),
        mesh=mesh,
        compiler_params=pltpu.CompilerParams(
            needs_layout_passes=False, use_tc_tiling_on_sc=False),
        scratch_types=[
            pltpu.VMEM((_PE,), jnp.int32),
            pltpu.VMEM((_PE,), jnp.int32),
            pltpu.VMEM((_PE,), jnp.float32),
            pltpu.VMEM((_NIDX, 128), jnp.int32),
            pltpu.VMEM((_PE, 16), jnp.float32),
            pltpu.VMEM((_MOL * 4,), jnp.float32),
            pltpu.VMEM((_MOL * 4,), jnp.float32),
            pltpu.VMEM((4, 16), jnp.float32),
            pltpu.VMEM((_PE * 4 // _MOL, _MOL), jnp.float32),
            pltpu.VMEM_SHARED((_AROWS, 16), jnp.float32),
        ],
    )


def _asm_body(acc_ref, pb_ref, w3_ref, b3_ref, out_ref, pred_ref, yscr):
    h = pl.program_id(0)
    x = acc_ref[0]
    out_ref[0, :, 0:_MOL] = x
    out_ref[0, :, _MOL:] = jnp.zeros((_MOL, _PRO - _MOL), jnp.float32)
    yscr[h, :] = jnp.sum(x, axis=0)

    @pl.when(h == _HEADS - 1)
    def _():
        pb = pb_ref[0]
        oh = (pb[:, None] == lax.broadcasted_iota(
            jnp.int32, (_MOL, _B), 1)).astype(jnp.float32)
        yb = jnp.dot(yscr[...], oh,
                     preferred_element_type=jnp.float32)
        pred = 0.01 * jnp.dot(yb.T, w3_ref[...],
                              preferred_element_type=jnp.float32)
        pred_ref[...] = pred + b3_ref[...]


def _assemble(acc8, pb512, w3, b3):
    return pl.pallas_call(
        _asm_body,
        grid=(_HEADS,),
        in_specs=[
            pl.BlockSpec((1, _MOL, _MOL), lambda h: (h, 0, 0)),
            pl.BlockSpec((1, _MOL), lambda h: (0, 0)),
            pl.BlockSpec((_HEADS, 1), lambda h: (0, 0)),
            pl.BlockSpec((1, 1), lambda h: (0, 0)),
        ],
        out_specs=[
            pl.BlockSpec((1, _MOL, _PRO), lambda h: (h, 0, 0)),
            pl.BlockSpec((_B, 1), lambda h: (0, 0)),
        ],
        out_shape=[
            jax.ShapeDtypeStruct((_HEADS, _MOL, _PRO), jnp.float32),
            jax.ShapeDtypeStruct((_B, 1), jnp.float32),
        ],
        scratch_shapes=[pltpu.VMEM((_HEADS, _MOL), jnp.float32)],
    )(acc8, pb512, w3, b3)


def kernel(mol_feats, pro_feats, pro_batch, bipartite_edge_index,
           bipartite_edge_attr, W1, b1, W3, b3):
    src = bipartite_edge_index[0]
    dst = bipartite_edge_index[1]
    dist = bipartite_edge_attr[:, 0]

    a_tab, p_tab = _prep(mol_feats, pro_feats[:_MOL],
                         W1[:_HID], W1[_HID:])
    a2 = a_tab.reshape(_MOL, 2, 4).transpose(1, 0, 2).reshape(2, _MOL * 4)
    p2 = p_tab.reshape(_MOL, 2, 4).transpose(1, 0, 2).reshape(2, _MOL * 4)
    b1_bc = jnp.broadcast_to(b1.reshape(2, 4, 1), (2, 4, 16))
    zrows = jnp.zeros((_PE, 16), jnp.float32)

    acc8 = _sc_scatter_fn()(src, dst, dist, a2, p2, b1_bc, zrows)

    inter, pred = _assemble(acc8, pro_batch[:_MOL].reshape(1, _MOL),
                            W3, b3.reshape(1, 1))
    return (inter, pred)
